# baseline (speedup 1.0000x reference)
import functools

import jax
import jax.numpy as jnp
import numpy as np
from jax import lax
from jax.experimental import pallas as pl
from jax.experimental.pallas import tpu as pltpu
from jax.experimental.pallas import tpu_sc as plsc

N_CELLS = 16384
N_GENES = 2048
LOGSCALE = np.float32(np.log10(10000.0))
LOGMAX = np.float32(np.log10(100000.0))
LO_P = N_CELLS // 4
HI_P = N_CELLS // 10
K2 = N_CELLS - HI_P
CB = 128
N_ITERS = 9


def _body(z_ref, zn_ref, acc_ref):
    i = pl.program_id(0)
    z = z_ref[:, :]
    zlog = jnp.log10(z)
    zn_ref[:, :] = jnp.tanh(zlog * np.float32(1.0 / LOGSCALE) - np.float32(1.0))

    x = zlog

    kf1 = jnp.float32(LO_P)
    kf2 = jnp.float32(K2)
    nf = jnp.float32(N_CELLS)

    ones_r = jnp.ones((1, N_CELLS), jnp.bfloat16)

    def rowcount(mask_bool):
        mb = mask_bool.astype(jnp.bfloat16)
        return jax.lax.dot_general(
            ones_r, mb, (((1,), (0,)), ((), ())),
            preferred_element_type=jnp.float32)

    lo0 = jnp.full((1, CB), -1e-3, jnp.float32)
    hi0 = jnp.full((1, CB), 4.001, jnp.float32)
    c0 = jnp.zeros((1, CB), jnp.float32)

    def it(_, carry):
        lo1, hi1, cl1, lo2, hi2, cl2 = carry
        m1 = (lo1 + hi1) * 0.5
        m2 = (lo2 + hi2) * 0.5
        mall = jnp.concatenate(
            [(x <= m1).astype(jnp.bfloat16), (x <= m2).astype(jnp.bfloat16)],
            axis=1)
        call = jax.lax.dot_general(
            ones_r, mall, (((1,), (0,)), ((), ())),
            preferred_element_type=jnp.float32)
        c1 = call[:, :CB]
        c2 = call[:, CB:]
        ge1 = c1 >= kf1
        ge2 = c2 >= kf2
        return (jnp.where(ge1, lo1, m1), jnp.where(ge1, m1, hi1),
                jnp.where(ge1, cl1, c1),
                jnp.where(ge2, lo2, m2), jnp.where(ge2, m2, hi2),
                jnp.where(ge2, cl2, c2))

    lo1, hi1, cl1, lo2, hi2, cl2 = jax.lax.fori_loop(
        0, N_ITERS, it, (lo0, hi0, c0, lo0, hi0, c0))

    mid1 = (lo1 + hi1) * 0.5
    mid2 = (lo2 + hi2) * 0.5

    sm1 = jnp.sum(jnp.minimum(x, lo1), axis=0, keepdims=True)
    sm2 = jnp.sum(jnp.maximum(x, lo2), axis=0, keepdims=True)

    bs1 = sm1 - (nf - cl1) * lo1 + (kf1 - cl1) * mid1
    ts = sm2 - cl2 * lo2 - (kf2 - cl2) * mid2

    b_tot = jnp.sum(bs1)
    t_tot = jnp.sum(ts)
    row = jax.lax.broadcasted_iota(jnp.int32, (8, 128), 0)
    col = jax.lax.broadcasted_iota(jnp.int32, (8, 128), 1)
    upd = jnp.where((row == 0) & (col == 0), b_tot,
                    jnp.where((row == 0) & (col == 1), t_tot,
                              jnp.float32(0.0)))

    @pl.when(i == 0)
    def _init():
        acc_ref[:, :] = jnp.zeros((8, 128), jnp.float32)

    acc_ref[:, :] += upd


def _sc_probe(Z):
    mesh = plsc.VectorSubcoreMesh(core_axis_name="c", subcore_axis_name="s")
    rows_per_w = N_CELLS // 32
    chunk = 8

    @functools.partial(
        pl.kernel, mesh=mesh,
        out_type=jax.ShapeDtypeStruct((16,), jnp.float32),
        scratch_types=[
            pltpu.VMEM((chunk, N_GENES), jnp.float32),
            pltpu.VMEM((16,), jnp.float32),
        ],
    )
    def k(z_hbm, out_hbm, buf, zv):
        wid = lax.axis_index("s") * 2 + lax.axis_index("c")
        base = wid * rows_per_w

        def body(i, carry):
            pltpu.sync_copy(z_hbm.at[pl.ds(base + i * chunk, chunk), :], buf)
            return carry

        lax.fori_loop(0, rows_per_w // chunk, body, 0)
        zv[...] = buf[0, pl.ds(0, 16)] * jnp.float32(0.0)
        @pl.when(wid == 0)
        def _():
            pltpu.sync_copy(zv, out_hbm)

    return k(Z)


@jax.jit
def kernel(Z):
    zn, acc = pl.pallas_call(
        _body,
        grid=(N_GENES // CB,),
        in_specs=[pl.BlockSpec((N_CELLS, CB), lambda i: (0, i))],
        out_specs=[
            pl.BlockSpec((N_CELLS, CB), lambda i: (0, i)),
            pl.BlockSpec((8, 128), lambda i: (0, 0)),
        ],
        out_shape=[
            jax.ShapeDtypeStruct((N_CELLS, N_GENES), jnp.float32),
            jax.ShapeDtypeStruct((8, 128), jnp.float32),
        ],
        compiler_params=pltpu.CompilerParams(
            dimension_semantics=("arbitrary",),
        ),
    )(Z)
    probe = _sc_probe(Z)
    lo = acc[0, 0] / np.float32(LO_P * N_GENES)
    hi = LOGMAX - acc[0, 1] / np.float32(HI_P * N_GENES)
    bit_cnst = (lo + hi + probe[0]).astype(jnp.float32)
    return zn, bit_cnst


# speedup vs baseline: 1.0665x; 1.0665x over previous
import functools

import jax
import jax.numpy as jnp
import numpy as np
from jax.experimental import pallas as pl
from jax.experimental.pallas import tpu as pltpu

N_CELLS = 16384
N_GENES = 2048
LOGSCALE = np.float32(np.log10(10000.0))
LOGMAX = np.float32(np.log10(100000.0))
LO_P = N_CELLS // 4
HI_P = N_CELLS // 10
K2 = N_CELLS - HI_P
CB = 128
N_ITERS = 9


def _body(z_ref, zn_ref, acc_ref):
    i = pl.program_id(0)
    z = z_ref[:, :]
    zlog = jnp.log10(z)
    zn_ref[:, :] = jnp.tanh(zlog * np.float32(1.0 / LOGSCALE) - np.float32(1.0))

    x = zlog

    kf1 = jnp.float32(LO_P)
    kf2 = jnp.float32(K2)
    nf = jnp.float32(N_CELLS)

    ones_r = jnp.ones((1, N_CELLS), jnp.bfloat16)

    def rowcount(mask_bool):
        mb = mask_bool.astype(jnp.bfloat16)
        return jax.lax.dot_general(
            ones_r, mb, (((1,), (0,)), ((), ())),
            preferred_element_type=jnp.float32)

    lo0 = jnp.full((1, CB), -1e-3, jnp.float32)
    hi0 = jnp.full((1, CB), 4.001, jnp.float32)
    c0 = jnp.zeros((1, CB), jnp.float32)

    def it(_, carry):
        lo1, hi1, cl1, lo2, hi2, cl2 = carry
        m1 = (lo1 + hi1) * 0.5
        m2 = (lo2 + hi2) * 0.5
        mall = jnp.concatenate(
            [(x <= m1).astype(jnp.bfloat16), (x <= m2).astype(jnp.bfloat16)],
            axis=1)
        call = jax.lax.dot_general(
            ones_r, mall, (((1,), (0,)), ((), ())),
            preferred_element_type=jnp.float32)
        c1 = call[:, :CB]
        c2 = call[:, CB:]
        ge1 = c1 >= kf1
        ge2 = c2 >= kf2
        return (jnp.where(ge1, lo1, m1), jnp.where(ge1, m1, hi1),
                jnp.where(ge1, cl1, c1),
                jnp.where(ge2, lo2, m2), jnp.where(ge2, m2, hi2),
                jnp.where(ge2, cl2, c2))

    lo1, hi1, cl1, lo2, hi2, cl2 = jax.lax.fori_loop(
        0, N_ITERS, it, (lo0, hi0, c0, lo0, hi0, c0))

    mid1 = (lo1 + hi1) * 0.5
    mid2 = (lo2 + hi2) * 0.5

    sm1 = jnp.sum(jnp.minimum(x, lo1), axis=0, keepdims=True)
    sm2 = jnp.sum(jnp.maximum(x, lo2), axis=0, keepdims=True)

    bs1 = sm1 - (nf - cl1) * lo1 + (kf1 - cl1) * mid1
    ts = sm2 - cl2 * lo2 - (kf2 - cl2) * mid2

    b_tot = jnp.sum(bs1)
    t_tot = jnp.sum(ts)
    row = jax.lax.broadcasted_iota(jnp.int32, (8, 128), 0)
    col = jax.lax.broadcasted_iota(jnp.int32, (8, 128), 1)
    upd = jnp.where((row == 0) & (col == 0), b_tot,
                    jnp.where((row == 0) & (col == 1), t_tot,
                              jnp.float32(0.0)))

    @pl.when(i == 0)
    def _init():
        acc_ref[:, :] = jnp.zeros((8, 128), jnp.float32)

    acc_ref[:, :] += upd


@jax.jit
def kernel(Z):
    zn, acc = pl.pallas_call(
        _body,
        grid=(N_GENES // CB,),
        in_specs=[pl.BlockSpec((N_CELLS, CB), lambda i: (0, i))],
        out_specs=[
            pl.BlockSpec((N_CELLS, CB), lambda i: (0, i)),
            pl.BlockSpec((8, 128), lambda i: (0, 0)),
        ],
        out_shape=[
            jax.ShapeDtypeStruct((N_CELLS, N_GENES), jnp.float32),
            jax.ShapeDtypeStruct((8, 128), jnp.float32),
        ],
        compiler_params=pltpu.CompilerParams(
            dimension_semantics=("arbitrary",),
        ),
    )(Z)
    lo = acc[0, 0] / np.float32(LO_P * N_GENES)
    hi = LOGMAX - acc[0, 1] / np.float32(HI_P * N_GENES)
    bit_cnst = (lo + hi).astype(jnp.float32)
    return zn, bit_cnst
